# Initial kernel scaffold; baseline (speedup 1.0000x reference)
import jax
import jax.numpy as jnp
from jax.experimental import pallas as pl


def kernel(inputs, noise_normal, rand_spike, rand_mult, noise_normal2):
    raise NotImplementedError("write your pallas kernel here")


# speedup vs baseline: 2.3658x; 2.3658x over previous
import jax
import jax.numpy as jnp
import numpy as np
from jax.experimental import pallas as pl

_INH_INC = 3000.0
_INH_REST = 500.0
_INH_TAU = 0.005
_NOISE_REST = 1000.0
_NOISE_TAU = 0.005
_NOISE_SIGMA = 50.0
_DT = 0.001
_INH_DECAY = float(np.exp(-_DT / _INH_TAU))
_NOISE_DECAY = float(np.exp(-_DT / _NOISE_TAU))
_LOG_DT = float(np.log(_DT))

_ROWS = 8
_LANE = 128


def _tri_scan_chunks(x3):
    u = (jax.lax.broadcasted_iota(jnp.int32, (_LANE, _LANE), 0)
         <= jax.lax.broadcasted_iota(jnp.int32, (_LANE, _LANE), 1)).astype(jnp.float32)
    return jax.lax.dot_general(
        x3, u, (((2,), (0,)), ((), ())),
        preferred_element_type=jnp.float32,
        precision=jax.lax.Precision.HIGHEST,
    )


def _body(x_ref, nn_ref, rs_ref, rm_ref, nn2_ref, out_ref, inh_ref, noise_ref):
    r, v = x_ref.shape
    nb = (v + _LANE - 1) // _LANE
    vp = nb * _LANE

    x = x_ref[...]
    ex = jnp.exp(x)
    s = jnp.sum(ex, axis=-1, keepdims=True)

    exp_pad = jnp.concatenate(
        [ex, jnp.zeros((r, vp - v), jnp.float32)], axis=-1
    ).reshape(r, nb, _LANE)

    cum1 = _tri_scan_chunks(exp_pad)
    csum = cum1[..., _LANE - 1]

    nb2 = (nb + _LANE - 1) // _LANE
    cs_pad = jnp.concatenate(
        [csum, jnp.zeros((r, nb2 * _LANE - nb), jnp.float32)], axis=-1
    ).reshape(r, nb2, _LANE)
    cum2 = _tri_scan_chunks(cs_pad)
    g = cum2[..., _LANE - 1]
    parts = [jnp.zeros((r, 1), jnp.float32)]
    for j in range(1, nb2):
        parts.append(parts[-1] + g[:, j - 1:j])
    excl_g = jnp.concatenate(parts, axis=-1)
    excl_chunk = (cum2 - cs_pad) + excl_g[..., None]
    excl_chunk = excl_chunk.reshape(r, nb2 * _LANE)[:, :nb]

    cum = cum1 + excl_chunk[..., None]
    cum_flat = cum.reshape(r, vp)[:, :v]

    t = rm_ref[...] * s
    mask = cum_flat >= t
    cnt = jnp.sum(mask.astype(jnp.int32), axis=-1, keepdims=True)
    idx = v - cnt
    idx = jnp.where(idx >= v, 0, idx)

    inh1 = jnp.float32(_INH_REST + (0.0 - _INH_REST) * _INH_DECAY)
    noise1 = _NOISE_REST + _NOISE_SIGMA * nn_ref[...]
    log_total = jnp.log(s) - inh1 + noise1
    spike = (jnp.log(rs_ref[...]) < log_total + _LOG_DT).astype(jnp.float32)

    iota = jax.lax.broadcasted_iota(jnp.int32, (r, v), 1)
    out_ref[...] = (iota == idx).astype(jnp.float32) * spike

    inh_ref[...] = _INH_REST + (inh1 - _INH_REST) * _INH_DECAY + spike * _INH_INC
    noise_ref[...] = (
        _NOISE_REST + (noise1 - _NOISE_REST) * _NOISE_DECAY
        + _NOISE_SIGMA * nn2_ref[...]
    )


@jax.jit
def kernel(inputs, noise_normal, rand_spike, rand_mult, noise_normal2):
    b, v = inputs.shape
    rm = rand_mult[:, None]
    row_spec = pl.BlockSpec((_ROWS, v), lambda i: (i, 0))
    col_spec = pl.BlockSpec((_ROWS, 1), lambda i: (i, 0))
    out_spikes, inh2, noise2 = pl.pallas_call(
        _body,
        grid=(b // _ROWS,),
        in_specs=[row_spec, col_spec, col_spec, col_spec, col_spec],
        out_specs=[row_spec, col_spec, col_spec],
        out_shape=[
            jax.ShapeDtypeStruct((b, v), inputs.dtype),
            jax.ShapeDtypeStruct((b, 1), inputs.dtype),
            jax.ShapeDtypeStruct((b, 1), inputs.dtype),
        ],
    )(inputs, noise_normal, rand_spike, rm, noise_normal2)
    return out_spikes, inh2, noise2
